# baseline (speedup 1.0000x reference)
import functools

import jax
import jax.numpy as jnp
from jax import lax
from jax.experimental import pallas as pl
from jax.experimental.pallas import tpu as pltpu
from jax.experimental.pallas import tpu_sc as plsc

N_DAYS = 365
N_STEPS = 48
N_IDS = 1024
BATCH = 1024

NC = 2
NS = 16
NW = NC * NS

TC_ROWS = 256
SC_ROWS = BATCH - TC_ROWS
BPW = SC_ROWS // NW
NBUF = 2


def _decode_body(cos_ref, sin_ref, idx_ref):
    two_pi = 2.0 * jnp.pi
    ang = jnp.arctan2(sin_ref[...], cos_ref[...])
    doy = jnp.round(jnp.mod(ang, two_pi) / two_pi * 365.0)
    idx_ref[...] = doy.astype(jnp.int32) - 1


def _decode_idx(x2):
    m = x2.reshape(BATCH, 2)
    cos8 = m[:, 0].reshape(8, BATCH // 8)
    sin8 = m[:, 1].reshape(8, BATCH // 8)
    idx8 = pl.pallas_call(
        _decode_body,
        out_shape=jax.ShapeDtypeStruct((8, BATCH // 8), jnp.int32),
    )(cos8, sin8)
    return idx8.reshape(BATCH)


def _gather_body(lut_hbm, idx_hbm, out_hbm, idx_v, *rest):
    bufs = rest[:NBUF]
    gsems = rest[NBUF:2 * NBUF]
    wsems = rest[2 * NBUF:3 * NBUF]

    wid = lax.axis_index("s") * NC + lax.axis_index("c")
    base = wid * BPW
    pltpu.sync_copy(idx_hbm.at[pl.ds(TC_ROWS + base, BPW)], idx_v)

    chunks = [idx_v[pl.ds(g * 8, 16)] for g in range(BPW // 8 - 1)]

    def day(b):
        g = min(b // 8, BPW // 8 - 2)
        return chunks[g][b - g * 8]

    def start_gather(u):
        s = u % NBUF
        return pltpu.async_copy(
            lut_hbm.at[pl.ds(day(u), 1)], bufs[s], gsems[s])

    def start_write(u):
        s = u % NBUF
        return pltpu.async_copy(
            bufs[s], out_hbm.at[pl.ds(base + u, 1)], wsems[s])

    pend_g = [None] * NBUF
    pend_w = [None] * NBUF
    for u in range(NBUF - 1):
        pend_g[u] = start_gather(u)
    for u in range(BPW):
        s = u % NBUF
        nxt = u + NBUF - 1
        if nxt < BPW:
            sn = nxt % NBUF
            if pend_w[sn] is not None:
                pend_w[sn].wait()
            pend_g[sn] = start_gather(nxt)
        pend_g[s].wait()
        pend_w[s] = start_write(u)
    for w in pend_w:
        if w is not None:
            w.wait()


_sc_gather = functools.partial(
    pl.kernel,
    out_type=jax.ShapeDtypeStruct((SC_ROWS, N_STEPS, N_IDS), jnp.float32),
    mesh=plsc.VectorSubcoreMesh(core_axis_name="c", subcore_axis_name="s",
                                num_cores=NC, num_subcores=NS),
    scratch_types=[
        pltpu.VMEM((BPW,), jnp.int32),
        *[pltpu.VMEM((1, N_STEPS, N_IDS), jnp.float32) for _ in range(NBUF)],
        *[pltpu.SemaphoreType.DMA for _ in range(2 * NBUF)],
    ],
    compiler_params=pltpu.CompilerParams(use_tc_tiling_on_sc=True),
)(_gather_body)


def _tc_copy_body(idx_ref, slab_ref, out_ref):
    del idx_ref
    out_ref[...] = slab_ref[...]


def _tc_gather(lut, idx_tc):
    grid_spec = pltpu.PrefetchScalarGridSpec(
        num_scalar_prefetch=1,
        grid=(TC_ROWS,),
        in_specs=[pl.BlockSpec((1, N_STEPS, N_IDS),
                               lambda i, idx_ref: (idx_ref[i], 0, 0))],
        out_specs=pl.BlockSpec((1, N_STEPS, N_IDS), lambda i, idx_ref: (i, 0, 0)),
    )
    return pl.pallas_call(
        _tc_copy_body,
        grid_spec=grid_spec,
        out_shape=jax.ShapeDtypeStruct((TC_ROWS, N_STEPS, N_IDS), jnp.float32),
    )(idx_tc, lut)


def kernel(x1, x2, lut):
    del x1
    idx = _decode_idx(x2)
    sc_out = _sc_gather(lut, idx)
    tc_out = _tc_gather(lut, idx[:TC_ROWS])
    return jnp.concatenate([tc_out, sc_out], axis=0)


# speedup vs baseline: 2.3651x; 2.3651x over previous
import functools

import jax
import jax.numpy as jnp
from jax import lax
from jax.experimental import pallas as pl
from jax.experimental.pallas import tpu as pltpu
from jax.experimental.pallas import tpu_sc as plsc

N_DAYS = 365
N_STEPS = 48
N_IDS = 1024
BATCH = 1024

NC = 2
NS = 16
NW = NC * NS
BPW = BATCH // NW


def _decode_body(cos_ref, sin_ref, idx_ref):
    two_pi = 2.0 * jnp.pi
    ang = jnp.arctan2(sin_ref[...], cos_ref[...])
    doy = jnp.round(jnp.mod(ang, two_pi) / two_pi * 365.0)
    idx_ref[...] = doy.astype(jnp.int32) - 1


def _decode_idx(x2):
    m = x2.reshape(BATCH, 2)
    cos8 = m[:, 0].reshape(8, BATCH // 8)
    sin8 = m[:, 1].reshape(8, BATCH // 8)
    idx8 = pl.pallas_call(
        _decode_body,
        out_shape=jax.ShapeDtypeStruct((8, BATCH // 8), jnp.int32),
    )(cos8, sin8)
    return idx8.reshape(BATCH)


def _gather_body(lut_hbm, day_hbm, row_hbm, ng_hbm, sel_hbm, out_hbm,
                 day_v, row_v, ng_v, sel_v,
                 buf0, buf1, gsem0, gsem1, wsem0, wsem1):
    wid = lax.axis_index("s") * NC + lax.axis_index("c")
    base = wid * BPW
    pltpu.sync_copy(day_hbm.at[pl.ds(base, BPW)], day_v)
    pltpu.sync_copy(row_hbm.at[pl.ds(base, BPW)], row_v)
    pltpu.sync_copy(ng_hbm.at[pl.ds(base, BPW)], ng_v)
    pltpu.sync_copy(sel_hbm.at[pl.ds(base, BPW)], sel_v)

    def lanes(v):
        return [v[pl.ds(g * 16, 16)] for g in range(BPW // 16)]

    day_c, row_c, ng_c, sel_c = lanes(day_v), lanes(row_v), lanes(ng_v), lanes(sel_v)

    def at(c, j):
        return c[j // 16][j % 16]

    bufs = (buf0, buf1)
    gsems = (gsem0, gsem1)
    wsems = (wsem0, wsem1)

    def wait_write(j):
        pltpu.make_async_copy(
            bufs[0], out_hbm.at[pl.ds(0, 1)], wsems[j % 2]).wait()

    for j in range(BPW):
        if j >= 2:
            wait_write(j - 2)
        d = at(day_c, j)
        r = at(row_c, j)
        new_run = at(ng_c, j) != 0
        sel = at(sel_c, j)
        for s in (0, 1):
            @pl.when(new_run & (sel == s))
            def _(s=s):
                pltpu.async_copy(lut_hbm.at[pl.ds(d, 1)], bufs[s], gsems[s])

            @pl.when(new_run & (sel == s))
            def _(s=s):
                pltpu.make_async_copy(
                    lut_hbm.at[pl.ds(d, 1)], bufs[s], gsems[s]).wait()

            @pl.when(sel == s)
            def _(s=s):
                pltpu.async_copy(
                    bufs[s], out_hbm.at[pl.ds(r, 1)], wsems[j % 2])
    wait_write(BPW - 2)
    wait_write(BPW - 1)


_sc_gather = functools.partial(
    pl.kernel,
    out_type=jax.ShapeDtypeStruct((BATCH, N_STEPS, N_IDS), jnp.float32),
    mesh=plsc.VectorSubcoreMesh(core_axis_name="c", subcore_axis_name="s",
                                num_cores=NC, num_subcores=NS),
    scratch_types=[
        *[pltpu.VMEM((BPW,), jnp.int32) for _ in range(4)],
        *[pltpu.VMEM((1, N_STEPS, N_IDS), jnp.float32) for _ in range(2)],
        *[pltpu.SemaphoreType.DMA for _ in range(4)],
    ],
    compiler_params=pltpu.CompilerParams(use_tc_tiling_on_sc=True),
)(_gather_body)


def kernel(x1, x2, lut):
    del x1
    idx = _decode_idx(x2)
    perm = jnp.argsort(idx).astype(jnp.int32)
    sday = jnp.take(idx, perm)
    pos = jnp.arange(BATCH, dtype=jnp.int32)
    prev = jnp.concatenate([sday[:1] - 1, sday[:-1]])
    ng = ((sday != prev) | (pos % BPW == 0)).astype(jnp.int32)
    runs = jnp.cumsum(ng).astype(jnp.int32)
    win_start = jnp.take(runs, (pos // BPW) * BPW)
    sel = (runs - win_start) % 2
    return _sc_gather(lut, sday, perm, ng, sel)


# speedup vs baseline: 2.6684x; 1.1282x over previous
import functools

import jax
import jax.numpy as jnp
from jax import lax
from jax.experimental import pallas as pl
from jax.experimental.pallas import tpu as pltpu
from jax.experimental.pallas import tpu_sc as plsc

N_DAYS = 365
N_STEPS = 48
N_IDS = 1024
BATCH = 1024

NC = 2
NS = 16
NW = NC * NS
BPW = BATCH // NW


def _decode_body(cos_ref, sin_ref, idx_ref):
    two_pi = 2.0 * jnp.pi
    ang = jnp.arctan2(sin_ref[...], cos_ref[...])
    doy = jnp.round(jnp.mod(ang, two_pi) / two_pi * 365.0)
    idx_ref[...] = doy.astype(jnp.int32) - 1


def _decode_idx(x2):
    m = x2.reshape(BATCH, 2)
    cos8 = m[:, 0].reshape(8, BATCH // 8)
    sin8 = m[:, 1].reshape(8, BATCH // 8)
    idx8 = pl.pallas_call(
        _decode_body,
        out_shape=jax.ShapeDtypeStruct((8, BATCH // 8), jnp.int32),
    )(cos8, sin8)
    return idx8.reshape(BATCH)


def _gather_body(lut_hbm, day_hbm, row_hbm, out_hbm,
                 day_v, row_v,
                 buf0, buf1, gsem0, gsem1, wsem0, wsem1):
    wid = lax.axis_index("s") * NC + lax.axis_index("c")
    base = wid * BPW
    pltpu.sync_copy(day_hbm.at[pl.ds(base, BPW)], day_v)
    pltpu.sync_copy(row_hbm.at[pl.ds(base, BPW)], row_v)

    def lanes(v):
        return [v[pl.ds(g * 16, 16)] for g in range(BPW // 16)]

    day_c, row_c = lanes(day_v), lanes(row_v)

    def at(c, j):
        return c[j // 16][j % 16]

    bufs = (buf0, buf1)
    gsems = (gsem0, gsem1)
    wsems = (wsem0, wsem1)

    def wait_write(j):
        pltpu.make_async_copy(
            bufs[0], out_hbm.at[pl.ds(0, 1)], wsems[j % 2]).wait()

    d_prev = at(day_c, 0) - 1
    run_cnt = jnp.int32(0)
    for j in range(BPW):
        if j >= 2:
            wait_write(j - 2)
        d = at(day_c, j)
        r = at(row_c, j)
        new_run = d != d_prev
        d_prev = d
        run_cnt = run_cnt + new_run.astype(jnp.int32)
        sel = lax.rem(run_cnt - 1, 2)
        for s in (0, 1):
            @pl.when(new_run & (sel == s))
            def _(s=s):
                pltpu.async_copy(lut_hbm.at[pl.ds(d, 1)], bufs[s], gsems[s])

            @pl.when(new_run & (sel == s))
            def _(s=s):
                pltpu.make_async_copy(
                    lut_hbm.at[pl.ds(d, 1)], bufs[s], gsems[s]).wait()

            @pl.when(sel == s)
            def _(s=s):
                pltpu.async_copy(
                    bufs[s], out_hbm.at[pl.ds(r, 1)], wsems[j % 2])
    wait_write(BPW - 2)
    wait_write(BPW - 1)


_sc_gather = functools.partial(
    pl.kernel,
    out_type=jax.ShapeDtypeStruct((BATCH, N_STEPS, N_IDS), jnp.float32),
    mesh=plsc.VectorSubcoreMesh(core_axis_name="c", subcore_axis_name="s",
                                num_cores=NC, num_subcores=NS),
    scratch_types=[
        *[pltpu.VMEM((BPW,), jnp.int32) for _ in range(2)],
        *[pltpu.VMEM((1, N_STEPS, N_IDS), jnp.float32) for _ in range(2)],
        *[pltpu.SemaphoreType.DMA for _ in range(4)],
    ],
    compiler_params=pltpu.CompilerParams(use_tc_tiling_on_sc=True),
)(_gather_body)


def kernel(x1, x2, lut):
    del x1
    idx = _decode_idx(x2)
    pos = jnp.arange(BATCH, dtype=jnp.int32)
    sday, perm = lax.sort((idx, pos), num_keys=1)
    return _sc_gather(lut, sday, perm)


# speedup vs baseline: 2.6821x; 1.0051x over previous
import functools

import jax
import jax.numpy as jnp
from jax import lax
from jax.experimental import pallas as pl
from jax.experimental.pallas import tpu as pltpu
from jax.experimental.pallas import tpu_sc as plsc

N_DAYS = 365
N_STEPS = 48
N_IDS = 1024
BATCH = 1024

NC = 2
NS = 16
NW = NC * NS
BPW = BATCH // NW


def _decode_body(cos_ref, sin_ref, idx_ref):
    two_pi = 2.0 * jnp.pi
    ang = jnp.arctan2(sin_ref[...], cos_ref[...])
    doy = jnp.round(jnp.mod(ang, two_pi) / two_pi * 365.0)
    pos = (lax.broadcasted_iota(jnp.int32, (8, BATCH // 8), 0) * (BATCH // 8)
           + lax.broadcasted_iota(jnp.int32, (8, BATCH // 8), 1))
    idx_ref[...] = (doy.astype(jnp.int32) - 1) * BATCH + pos


def _decode_idx(x2):
    m = x2.reshape(BATCH, 2)
    cos8 = m[:, 0].reshape(8, BATCH // 8)
    sin8 = m[:, 1].reshape(8, BATCH // 8)
    idx8 = pl.pallas_call(
        _decode_body,
        out_shape=jax.ShapeDtypeStruct((8, BATCH // 8), jnp.int32),
    )(cos8, sin8)
    return idx8.reshape(BATCH)


def _gather_body(lut_hbm, packed_hbm, out_hbm,
                 packed_v, buf0, buf1, gsem0, gsem1, wsem0, wsem1):
    wid = lax.axis_index("s") * NC + lax.axis_index("c")
    base = wid * BPW
    pltpu.sync_copy(packed_hbm.at[pl.ds(base, BPW)], packed_v)

    packed_c = [packed_v[pl.ds(g * 16, 16)] for g in range(BPW // 16)]

    def at(c, j):
        return c[j // 16][j % 16]

    bufs = (buf0, buf1)
    gsems = (gsem0, gsem1)
    wsems = (wsem0, wsem1)

    def wait_write(j):
        pltpu.make_async_copy(
            bufs[0], out_hbm.at[pl.ds(0, 1)], wsems[j % 2]).wait()

    d_prev = lax.shift_right_logical(at(packed_c, 0), 10) - 1
    run_cnt = jnp.int32(0)
    for j in range(BPW):
        if j >= 2:
            wait_write(j - 2)
        p = at(packed_c, j)
        d = lax.shift_right_logical(p, 10)
        r = lax.bitwise_and(p, BATCH - 1)
        new_run = d != d_prev
        d_prev = d
        run_cnt = run_cnt + new_run.astype(jnp.int32)
        sel = lax.rem(run_cnt - 1, 2)
        for s in (0, 1):
            @pl.when(new_run & (sel == s))
            def _(s=s):
                pltpu.async_copy(lut_hbm.at[pl.ds(d, 1)], bufs[s], gsems[s])

            @pl.when(new_run & (sel == s))
            def _(s=s):
                pltpu.make_async_copy(
                    lut_hbm.at[pl.ds(d, 1)], bufs[s], gsems[s]).wait()

            @pl.when(sel == s)
            def _(s=s):
                pltpu.async_copy(
                    bufs[s], out_hbm.at[pl.ds(r, 1)], wsems[j % 2])
    wait_write(BPW - 2)
    wait_write(BPW - 1)


_sc_gather = functools.partial(
    pl.kernel,
    out_type=jax.ShapeDtypeStruct((BATCH, N_STEPS, N_IDS), jnp.float32),
    mesh=plsc.VectorSubcoreMesh(core_axis_name="c", subcore_axis_name="s",
                                num_cores=NC, num_subcores=NS),
    scratch_types=[
        pltpu.VMEM((BPW,), jnp.int32),
        *[pltpu.VMEM((1, N_STEPS, N_IDS), jnp.float32) for _ in range(2)],
        *[pltpu.SemaphoreType.DMA for _ in range(4)],
    ],
    compiler_params=pltpu.CompilerParams(use_tc_tiling_on_sc=True),
)(_gather_body)


def kernel(x1, x2, lut):
    del x1
    packed = _decode_idx(x2)
    return _sc_gather(lut, lax.sort(packed))
